# baseline (speedup 1.0000x reference)
import functools

import jax
import jax.numpy as jnp
from jax.experimental import pallas as pl
from jax.experimental.pallas import tpu as pltpu

N = 4800
H = 128
NG = 8
NPG = 600
NJ = 30
BM = 480
NB = N // BM


def _bn(t, g, b):
    mu = jnp.mean(t, axis=0, keepdims=True)
    var = jnp.mean((t - mu) * (t - mu), axis=0, keepdims=True)
    return g * (t - mu) / jnp.sqrt(var + 1e-5) + b


def _fused(x_ref, gp_ref, cand_ref, mask_ref, adj_ref,
           w01_ref, b01_ref, g01_ref, be01_ref, w02_ref, b02_ref, g02_ref, be02_ref,
           w11_ref, b11_ref, g11_ref, be11_ref, w12_ref, b12_ref, g12_ref, be12_ref,
           aW1_ref, ab1_ref, aW2_ref, ab2_ref, cW1_ref, cb1_ref, cW2_ref, cb2_ref,
           pi_ref, v_ref,
           t0_scr, hw_scr, hwlo_scr, t1_scr):
    p = pl.program_id(0)
    i = pl.program_id(1)
    blk = adj_ref[...]

    @pl.when(p == 0)
    def _phase_a():
        pooled0 = jnp.dot(blk, x_ref[...], preferred_element_type=jnp.float32)
        t0 = jnp.dot(pooled0, w01_ref[...],
                     preferred_element_type=jnp.float32) + b01_ref[...]
        t0_scr[pl.ds(i * BM, BM), :] = t0

    @pl.when((p == 1) & (i == 0))
    def _epilogue_a():
        t = t0_scr[...]
        h = jnp.maximum(_bn(t, g01_ref[...], be01_ref[...]), 0.0)
        t2 = jnp.dot(h, w02_ref[...],
                     preferred_element_type=jnp.float32) + b02_ref[...]
        h1 = jnp.maximum(_bn(t2, g02_ref[...], be02_ref[...]), 0.0)
        hw = jnp.dot(h1, w11_ref[...], preferred_element_type=jnp.float32)
        hw_hi = hw.astype(jnp.bfloat16)
        hw_scr[...] = hw_hi
        hwlo_scr[...] = (hw - hw_hi.astype(jnp.float32)).astype(jnp.bfloat16)

    @pl.when(p == 1)
    def _phase_b():
        blk_bf = blk.astype(jnp.bfloat16)
        t1_scr[pl.ds(i * BM, BM), :] = (
            jnp.dot(blk_bf, hw_scr[...], preferred_element_type=jnp.float32)
            + jnp.dot(blk_bf, hwlo_scr[...], preferred_element_type=jnp.float32))

    @pl.when((p == 1) & (i == NB - 1))
    def _epilogue_b():
        t1 = t1_scr[...] + b11_ref[...]
        h = jnp.maximum(_bn(t1, g11_ref[...], be11_ref[...]), 0.0)
        t2 = jnp.dot(h, w12_ref[...],
                     preferred_element_type=jnp.float32) + b12_ref[...]
        h2 = jnp.maximum(_bn(t2, g12_ref[...], be12_ref[...]), 0.0)

        hp_all = jnp.dot(gp_ref[...], h2,
                         preferred_element_type=jnp.float32)
        v = jnp.dot(jnp.tanh(jnp.dot(hp_all, cW1_ref[...],
                                     preferred_element_type=jnp.float32)
                             + cb1_ref[...]),
                    cW2_ref[...], preferred_element_type=jnp.float32) \
            + cb2_ref[...]
        v_ref[...] = v

        for g in range(NG):
            seg = jax.lax.slice(h2, (g * NPG, 0), ((g + 1) * NPG, H))
            cand_g = cand_ref[g]
            onehot = (jax.lax.broadcasted_iota(jnp.int32, (NJ, NPG), 1)
                      == cand_g).astype(jnp.float32)
            cf = jnp.dot(onehot, seg,
                         preferred_element_type=jnp.float32)
            hp_g = jax.lax.slice(hp_all, (g, 0), (g + 1, H))
            feat = jnp.concatenate(
                [cf, jnp.broadcast_to(hp_g, (NJ, H))], axis=1)
            tg = jnp.tanh(jnp.dot(feat, aW1_ref[...],
                                  preferred_element_type=jnp.float32)
                          + ab1_ref[...])
            sg = jnp.dot(tg, aW2_ref[...],
                         preferred_element_type=jnp.float32) + ab2_ref[...]
            mg = mask_ref[g]
            sg = jnp.where(mg != 0.0, -jnp.inf, sg)
            mx = jnp.max(sg, axis=0, keepdims=True)
            e = jnp.exp(sg - mx)
            pi_ref[g] = e / jnp.sum(e, axis=0, keepdims=True)


def _forward(x, graph_pool, adj, candidate, mask_f, ws, interpret=False):
    (w01, b01, g01, be01, w02, b02, g02, be02,
     w11, b11, g11, be11, w12, b12, g12, be12,
     aW1, ab1, aW2, ab2, cW1, cb1, cW2, cb2) = ws

    def row2(a):
        return a.reshape(1, -1)

    full = lambda shape: pl.BlockSpec(shape, lambda p, i: tuple(0 for _ in shape))
    in_specs = [
        full((N, 2)),
        full((NG, N)),
        full((NG, NJ, 1)),
        full((NG, NJ, 1)),
        pl.BlockSpec((BM, N), lambda p, i: (i, 0)),
    ]
    weights = [w01, row2(b01), row2(g01), row2(be01),
               w02, row2(b02), row2(g02), row2(be02),
               w11, row2(b11), row2(g11), row2(be11),
               w12, row2(b12), row2(g12), row2(be12),
               aW1, row2(ab1), aW2, row2(ab2),
               cW1, row2(cb1), cW2, row2(cb2)]
    in_specs += [full(w.shape) for w in weights]

    pi, v = pl.pallas_call(
        _fused,
        grid=(2, NB),
        in_specs=in_specs,
        out_specs=[full((NG, NJ, 1)), full((NG, 1))],
        out_shape=[jax.ShapeDtypeStruct((NG, NJ, 1), jnp.float32),
                   jax.ShapeDtypeStruct((NG, 1), jnp.float32)],
        scratch_shapes=[pltpu.VMEM((N, H), jnp.float32),
                        pltpu.VMEM((N, H), jnp.bfloat16),
                        pltpu.VMEM((N, H), jnp.bfloat16),
                        pltpu.VMEM((N, H), jnp.float32)],
        compiler_params=pltpu.CompilerParams(
            dimension_semantics=("arbitrary", "arbitrary")),
        interpret=interpret,
    )(x, graph_pool, candidate.reshape(NG, NJ, 1).astype(jnp.int32),
      mask_f.reshape(NG, NJ, 1), adj, *weights)
    return pi, v


def kernel(x, n_j, graph_pool, padded_nei, adj, candidate, mask,
           m0_W1, m0_b1, m0_g1, m0_be1, m0_W2, m0_b2, m0_g2, m0_be2,
           m1_W1, m1_b1, m1_g1, m1_be1, m1_W2, m1_b2, m1_g2, m1_be2,
           aW1, ab1, aW2, ab2, cW1, cb1, cW2, cb2):
    ws = (m0_W1, m0_b1, m0_g1, m0_be1, m0_W2, m0_b2, m0_g2, m0_be2,
          m1_W1, m1_b1, m1_g1, m1_be1, m1_W2, m1_b2, m1_g2, m1_be2,
          aW1, ab1, aW2, ab2, cW1, cb1, cW2, cb2)
    pi, v = _forward(x, graph_pool, adj, candidate,
                     mask.astype(jnp.float32), ws)
    return (pi, v)


# speedup vs baseline: 1.0860x; 1.0860x over previous
import functools

import jax
import jax.numpy as jnp
from jax.experimental import pallas as pl
from jax.experimental.pallas import tpu as pltpu

N = 4800
H = 128
NG = 8
NPG = 600
NJ = 30
BM = 600
NB = N // BM


def _bn(t, g, b):
    mu = jnp.mean(t, axis=0, keepdims=True)
    var = jnp.mean((t - mu) * (t - mu), axis=0, keepdims=True)
    return g * (t - mu) / jnp.sqrt(var + 1e-5) + b


def _fused(x_ref, gp_ref, cand_ref, mask_ref, adj_ref,
           w01_ref, b01_ref, g01_ref, be01_ref, w02_ref, b02_ref, g02_ref, be02_ref,
           w11_ref, b11_ref, g11_ref, be11_ref, w12_ref, b12_ref, g12_ref, be12_ref,
           aW1_ref, ab1_ref, aW2_ref, ab2_ref, cW1_ref, cb1_ref, cW2_ref, cb2_ref,
           pi_ref, v_ref,
           t0_scr, hw_scr, t1_scr):
    p = pl.program_id(0)
    i = pl.program_id(1)
    blk = adj_ref[...]

    @pl.when(p == 0)
    def _phase_a():
        pooled0 = jnp.dot(blk, x_ref[...], preferred_element_type=jnp.float32)
        t0 = jnp.dot(pooled0, w01_ref[...],
                     preferred_element_type=jnp.float32) + b01_ref[...]
        t0_scr[pl.ds(i * BM, BM), :] = t0

    @pl.when((p == 1) & (i == 0))
    def _epilogue_a():
        t = t0_scr[...]
        h = jnp.maximum(_bn(t, g01_ref[...], be01_ref[...]), 0.0)
        t2 = jnp.dot(h, w02_ref[...],
                     preferred_element_type=jnp.float32) + b02_ref[...]
        h1 = jnp.maximum(_bn(t2, g02_ref[...], be02_ref[...]), 0.0)
        hw_scr[...] = jnp.dot(h1, w11_ref[...],
                              preferred_element_type=jnp.float32)

    @pl.when(p == 1)
    def _phase_b():
        t1_scr[pl.ds(i * BM, BM), :] = jnp.dot(
            blk, hw_scr[...], preferred_element_type=jnp.float32)

    @pl.when((p == 1) & (i == NB - 1))
    def _epilogue_b():
        t1 = t1_scr[...] + b11_ref[...]
        h = jnp.maximum(_bn(t1, g11_ref[...], be11_ref[...]), 0.0)
        t2 = jnp.dot(h, w12_ref[...],
                     preferred_element_type=jnp.float32) + b12_ref[...]
        h2 = jnp.maximum(_bn(t2, g12_ref[...], be12_ref[...]), 0.0)

        hp_all = jnp.dot(gp_ref[...], h2,
                         preferred_element_type=jnp.float32)
        v = jnp.dot(jnp.tanh(jnp.dot(hp_all, cW1_ref[...],
                                     preferred_element_type=jnp.float32)
                             + cb1_ref[...]),
                    cW2_ref[...], preferred_element_type=jnp.float32) \
            + cb2_ref[...]
        v_ref[...] = v

        for g in range(NG):
            seg = jax.lax.slice(h2, (g * NPG, 0), ((g + 1) * NPG, H))
            cand_g = cand_ref[g]
            onehot = (jax.lax.broadcasted_iota(jnp.int32, (NJ, NPG), 1)
                      == cand_g).astype(jnp.float32)
            cf = jnp.dot(onehot, seg,
                         preferred_element_type=jnp.float32)
            hp_g = jax.lax.slice(hp_all, (g, 0), (g + 1, H))
            feat = jnp.concatenate(
                [cf, jnp.broadcast_to(hp_g, (NJ, H))], axis=1)
            tg = jnp.tanh(jnp.dot(feat, aW1_ref[...],
                                  preferred_element_type=jnp.float32)
                          + ab1_ref[...])
            sg = jnp.dot(tg, aW2_ref[...],
                         preferred_element_type=jnp.float32) + ab2_ref[...]
            mg = mask_ref[g]
            sg = jnp.where(mg != 0.0, -jnp.inf, sg)
            mx = jnp.max(sg, axis=0, keepdims=True)
            e = jnp.exp(sg - mx)
            pi_ref[g] = e / jnp.sum(e, axis=0, keepdims=True)


def _forward(x, graph_pool, adj, candidate, mask_f, ws, interpret=False):
    (w01, b01, g01, be01, w02, b02, g02, be02,
     w11, b11, g11, be11, w12, b12, g12, be12,
     aW1, ab1, aW2, ab2, cW1, cb1, cW2, cb2) = ws

    def row2(a):
        return a.reshape(1, -1)

    full = lambda shape: pl.BlockSpec(shape, lambda p, i: tuple(0 for _ in shape))
    in_specs = [
        full((N, 2)),
        full((NG, N)),
        full((NG, NJ, 1)),
        full((NG, NJ, 1)),
        pl.BlockSpec((BM, N), lambda p, i: (i, 0)),
    ]
    weights = [w01, row2(b01), row2(g01), row2(be01),
               w02, row2(b02), row2(g02), row2(be02),
               w11, row2(b11), row2(g11), row2(be11),
               w12, row2(b12), row2(g12), row2(be12),
               aW1, row2(ab1), aW2, row2(ab2),
               cW1, row2(cb1), cW2, row2(cb2)]
    in_specs += [full(w.shape) for w in weights]

    pi, v = pl.pallas_call(
        _fused,
        grid=(2, NB),
        in_specs=in_specs,
        out_specs=[full((NG, NJ, 1)), full((NG, 1))],
        out_shape=[jax.ShapeDtypeStruct((NG, NJ, 1), jnp.float32),
                   jax.ShapeDtypeStruct((NG, 1), jnp.float32)],
        scratch_shapes=[pltpu.VMEM((N, H), jnp.float32),
                        pltpu.VMEM((N, H), jnp.float32),
                        pltpu.VMEM((N, H), jnp.float32)],
        compiler_params=pltpu.CompilerParams(
            dimension_semantics=("arbitrary", "arbitrary")),
        interpret=interpret,
    )(x, graph_pool, candidate.reshape(NG, NJ, 1).astype(jnp.int32),
      mask_f.reshape(NG, NJ, 1), adj, *weights)
    return pi, v


def kernel(x, n_j, graph_pool, padded_nei, adj, candidate, mask,
           m0_W1, m0_b1, m0_g1, m0_be1, m0_W2, m0_b2, m0_g2, m0_be2,
           m1_W1, m1_b1, m1_g1, m1_be1, m1_W2, m1_b2, m1_g2, m1_be2,
           aW1, ab1, aW2, ab2, cW1, cb1, cW2, cb2):
    ws = (m0_W1, m0_b1, m0_g1, m0_be1, m0_W2, m0_b2, m0_g2, m0_be2,
          m1_W1, m1_b1, m1_g1, m1_be1, m1_W2, m1_b2, m1_g2, m1_be2,
          aW1, ab1, aW2, ab2, cW1, cb1, cW2, cb2)
    pi, v = _forward(x, graph_pool, adj, candidate,
                     mask.astype(jnp.float32), ws)
    return (pi, v)
